# baseline (speedup 1.0000x reference)
import functools

import jax
import jax.numpy as jnp
from jax import lax
from jax.experimental import pallas as pl
from jax.experimental.pallas import tpu as pltpu
from jax.experimental.pallas import tpu_sc as plsc

NB = 4


@functools.lru_cache(maxsize=None)
def _build(bsz, seq, v, d):
    info = plsc.get_sparse_core_info()
    nw = info.num_cores * info.num_subcores
    BB = bsz // nw
    assert BB == 128
    n_chunks = seq
    assert n_chunks % NB == 0

    mesh = plsc.VectorSubcoreMesh(core_axis_name="c", subcore_axis_name="s")

    @functools.partial(
        pl.kernel,
        mesh=mesh,
        out_type=jax.ShapeDtypeStruct((seq, d, bsz), jnp.float32),
        scratch_types=[
            pltpu.VMEM((NB, BB), jnp.int32),
            pltpu.VMEM((NB, BB, 128), jnp.float32),
            pltpu.VMEM((NB, d, BB), jnp.float32),
            [pltpu.SemaphoreType.DMA] * NB,
            [pltpu.SemaphoreType.DMA] * NB,
            [pltpu.SemaphoreType.DMA] * NB,
        ],
        compiler_params=pltpu.CompilerParams(
            use_tc_tiling_on_sc=True, needs_layout_passes=False),
    )
    def k(table_hbm, ids_hbm, out_hbm, idx, gin, gout, sem_i, sem_g, sem_s):
        wid = lax.axis_index("s") * info.num_cores + lax.axis_index("c")
        b0 = wid * BB
        it16 = lax.iota(jnp.int32, 16)
        rowvecs_bb = [it16 + (c * 16) for c in range(BB // 16)]

        def idx_copy(i, b):
            return pltpu.make_async_copy(
                ids_hbm.at[i, pl.ds(b0, BB)], idx.at[b], sem_i[b])

        def gather_copy(b):
            return pltpu.make_async_copy(
                table_hbm.at[idx.at[b]], gin.at[b], sem_g[b])

        def store_copy(i, b):
            return pltpu.make_async_copy(
                gout.at[b], out_hbm.at[i, :, pl.ds(b0, BB)], sem_s[b])

        def transpose(b):
            nc = BB // 16
            for e in range(d):
                evec = it16 * 0 + e
                grp = []
                for c in range(nc):
                    grp.append(plsc.load_gather(
                        gin.at[b], [rowvecs_bb[c], evec]))
                for c in range(nc):
                    gout[b, e, pl.ds(c * 16, 16)] = grp[c]

        for j in range(NB):
            idx_copy(j, j).start()
        idx_copy(0, 0).wait()
        gather_copy(0).start()
        idx_copy(1, 1).wait()
        gather_copy(1).start()

        def body(g, carry):
            for b in range(NB):
                i = NB * g + b

                gather_copy(b).wait()

                @pl.when(i + 2 < n_chunks)
                def _():
                    b2 = (b + 2) % NB
                    idx_copy(i + 2, b2).wait()
                    gather_copy(b2).start()

                @pl.when(i + NB < n_chunks)
                def _():
                    idx_copy(i + NB, b).start()

                @pl.when(i >= NB)
                def _():
                    store_copy(i - NB, b).wait()

                transpose(b)
                store_copy(i, b).start()
            return carry

        lax.fori_loop(0, n_chunks // NB, body, 0)
        for j in range(NB):
            store_copy(n_chunks - NB + j, j).wait()

    return k


def kernel(input_ids, table):
    bsz, seq = input_ids.shape
    v, d = table.shape
    ids_t = input_ids.T.astype(jnp.int32)
    table_pad = jnp.pad(table, ((0, 0), (0, 128 - d)))
    out_t = _build(bsz, seq, v, d)(table_pad, ids_t)
    return (out_t.transpose(2, 0, 1),)


# speedup vs baseline: 2.1859x; 2.1859x over previous
import functools

import jax
import jax.numpy as jnp
from jax import lax
from jax.experimental import pallas as pl
from jax.experimental.pallas import tpu as pltpu
from jax.experimental.pallas import tpu_sc as plsc

NB = 4


@functools.lru_cache(maxsize=None)
def _build(bsz, seq, v, d):
    info = plsc.get_sparse_core_info()
    nw = info.num_cores * info.num_subcores
    BB = bsz // nw
    assert BB == 128
    n_chunks = seq
    assert n_chunks % NB == 0

    mesh = plsc.VectorSubcoreMesh(core_axis_name="c", subcore_axis_name="s")

    @functools.partial(
        pl.kernel,
        mesh=mesh,
        out_type=jax.ShapeDtypeStruct((seq, d, bsz), jnp.float32),
        scratch_types=[
            pltpu.VMEM((NB, BB), jnp.int32),
            pltpu.VMEM((NB, BB, 128), jnp.float32),
            pltpu.VMEM((NB, d, BB), jnp.float32),
            [pltpu.SemaphoreType.DMA] * NB,
            [pltpu.SemaphoreType.DMA] * NB,
            [pltpu.SemaphoreType.DMA] * NB,
        ],
        compiler_params=pltpu.CompilerParams(
            use_tc_tiling_on_sc=True, needs_layout_passes=False),
    )
    def k(table_hbm, ids_hbm, out_hbm, idx, gin, gout, sem_i, sem_g, sem_s):
        wid = lax.axis_index("s") * info.num_cores + lax.axis_index("c")
        b0 = wid * BB
        it16 = lax.iota(jnp.int32, 16)
        rotvecs = [(it16 + k) & 15 for k in range(16)]

        def idx_copy(i, b):
            return pltpu.make_async_copy(
                ids_hbm.at[i, pl.ds(b0, BB)], idx.at[b], sem_i[b])

        def gather_copy(b):
            return pltpu.make_async_copy(
                table_hbm.at[idx.at[b]], gin.at[b], sem_g[b])

        def store_copy(i, b):
            return pltpu.make_async_copy(
                gout.at[b], out_hbm.at[i, :, pl.ds(b0, BB)], sem_s[b])

        def transpose(b):
            gi, go = gin.at[b], gout.at[b]
            ne = d // 16

            def tbody(t, carry):
                r0 = (t // ne) * 16
                e0 = (t % ne) * 16
                rvec = it16 + r0
                for k0 in range(0, 16, 4):
                    evs = [rotvecs[k0 + j] + e0 for j in range(4)]
                    vals = [plsc.load_gather(gi, [rvec, evs[j]])
                            for j in range(4)]
                    for j in range(4):
                        plsc.store_scatter(go, [evs[j], rvec], vals[j])
                return carry

            lax.fori_loop(0, (BB // 16) * ne, tbody, 0)

        for j in range(NB):
            idx_copy(j, j).start()
        idx_copy(0, 0).wait()
        gather_copy(0).start()
        idx_copy(1, 1).wait()
        gather_copy(1).start()

        def body(g, carry):
            for b in range(NB):
                i = NB * g + b

                gather_copy(b).wait()

                @pl.when(i + 2 < n_chunks)
                def _():
                    b2 = (b + 2) % NB
                    idx_copy(i + 2, b2).wait()
                    gather_copy(b2).start()

                @pl.when(i + NB < n_chunks)
                def _():
                    idx_copy(i + NB, b).start()

                @pl.when(i >= NB)
                def _():
                    store_copy(i - NB, b).wait()

                transpose(b)
                store_copy(i, b).start()
            return carry

        lax.fori_loop(0, n_chunks // NB, body, 0)
        for j in range(NB):
            store_copy(n_chunks - NB + j, j).wait()

    return k


def kernel(input_ids, table):
    bsz, seq = input_ids.shape
    v, d = table.shape
    ids_t = input_ids.T.astype(jnp.int32)
    table_pad = jnp.pad(table, ((0, 0), (0, 128 - d)))
    out_t = _build(bsz, seq, v, d)(table_pad, ids_t)
    return (out_t.transpose(2, 0, 1),)


# speedup vs baseline: 2.9385x; 1.3443x over previous
import functools

import jax
import jax.numpy as jnp
from jax import lax
from jax.experimental import pallas as pl
from jax.experimental.pallas import tpu as pltpu
from jax.experimental.pallas import tpu_sc as plsc

NB = 4


@functools.lru_cache(maxsize=None)
def _build(bsz, seq, v, d):
    info = plsc.get_sparse_core_info()
    nw = info.num_cores * info.num_subcores
    BB = bsz // nw
    assert BB == 128
    n_chunks = seq
    assert n_chunks % NB == 0

    mesh = plsc.VectorSubcoreMesh(core_axis_name="c", subcore_axis_name="s")

    @functools.partial(
        pl.kernel,
        mesh=mesh,
        out_type=jax.ShapeDtypeStruct((seq, d, bsz), jnp.float32),
        scratch_types=[
            pltpu.VMEM((NB, BB), jnp.int32),
            pltpu.VMEM((NB, BB, 128), jnp.float32),
            pltpu.VMEM((NB, d, BB), jnp.float32),
            [pltpu.SemaphoreType.DMA] * NB,
            [pltpu.SemaphoreType.DMA] * NB,
            [pltpu.SemaphoreType.DMA] * NB,
        ],
        compiler_params=pltpu.CompilerParams(
            use_tc_tiling_on_sc=True, needs_layout_passes=False),
    )
    def k(table_hbm, ids_hbm, out_hbm, idx, gin, gout, sem_i, sem_g, sem_s):
        wid = lax.axis_index("s") * info.num_cores + lax.axis_index("c")
        b0 = wid * BB
        it16 = lax.iota(jnp.int32, 16)
        rotvecs = [(it16 + k) & 15 for k in range(16)]

        def idx_copy(i, b):
            return pltpu.make_async_copy(
                ids_hbm.at[i, pl.ds(b0, BB)], idx.at[b], sem_i[b])

        def gather_copy(b):
            return pltpu.make_async_copy(
                table_hbm.at[idx.at[b]], gin.at[b], sem_g[b])

        def store_copy(i, b):
            return pltpu.make_async_copy(
                gout.at[b], out_hbm.at[i, :, pl.ds(b0, BB)], sem_s[b])

        def transpose(b):
            gi, go = gin.at[b], gout.at[b]
            ne = d // 16

            def tbody(t, carry):
                r0 = (t // ne) * 16
                e0 = (t % ne) * 16
                rvec = it16 + r0
                for k0 in range(0, 16, 4):
                    evs = [rotvecs[k0 + j] + e0 for j in range(4)]
                    vals = [plsc.load_gather(gi, [rvec, evs[j]])
                            for j in range(4)]
                    for j in range(4):
                        plsc.store_scatter(go, [evs[j], rvec], vals[j])
                return carry

            lax.fori_loop(0, (BB // 16) * ne, tbody, 0)

        for j in range(NB):
            idx_copy(j, j).start()
        idx_copy(0, 0).wait()
        gather_copy(0).start()
        idx_copy(1, 1).wait()
        gather_copy(1).start()

        def body(g, carry):
            for b in range(NB):
                i = NB * g + b

                gather_copy(b).wait()

                @pl.when(i + 2 < n_chunks)
                def _():
                    b2 = (b + 2) % NB
                    idx_copy(i + 2, b2).wait()
                    gather_copy(b2).start()

                @pl.when(i + NB < n_chunks)
                def _():
                    idx_copy(i + NB, b).start()

                @pl.when(i >= NB)
                def _():
                    store_copy(i - NB, b).wait()

                transpose(b)
                store_copy(i, b).start()
            return carry

        lax.fori_loop(0, n_chunks // NB, body, 0)
        for j in range(NB):
            store_copy(n_chunks - NB + j, j).wait()

    return k


@functools.lru_cache(maxsize=None)
def _build_pad(v, d):
    info = plsc.get_sparse_core_info()
    nw = info.num_cores * info.num_subcores
    n_units = (v + 127) // 128
    vtail = v % 128
    n_t = (n_units + nw - 1) // nw

    mesh = plsc.VectorSubcoreMesh(core_axis_name="c", subcore_axis_name="s")

    @functools.partial(
        pl.kernel,
        mesh=mesh,
        out_type=jax.ShapeDtypeStruct((v, 128), jnp.float32),
        scratch_types=[
            pltpu.VMEM((2, d, 128), jnp.float32),
            pltpu.VMEM((2, 128, 128), jnp.float32),
            pltpu.VMEM((vtail, 128), jnp.float32),
            [pltpu.SemaphoreType.DMA] * 2,
            [pltpu.SemaphoreType.DMA] * 2,
            pltpu.SemaphoreType.DMA,
        ],
        compiler_params=pltpu.CompilerParams(
            use_tc_tiling_on_sc=True, needs_layout_passes=False),
    )
    def k1(tin_hbm, tail_hbm, out_hbm, gin, gout, gtail, sem_g, sem_s, sem_t):
        wid = lax.axis_index("s") * info.num_cores + lax.axis_index("c")
        it16 = lax.iota(jnp.int32, 16)
        rotvecs = [(it16 + kk) & 15 for kk in range(16)]

        def unit(t):
            return t * nw + wid

        def in_copy(t, b):
            return pltpu.make_async_copy(
                tin_hbm.at[:, pl.ds(unit(t) * 128, 128)], gin.at[b], sem_g[b])

        def store_copy(t, b):
            return pltpu.make_async_copy(
                gout.at[b], out_hbm.at[pl.ds(unit(t) * 128, 128)], sem_s[b])

        def transpose(gi, go, ncol):
            ne = d // 16

            def tbody(tt, carry):
                r0 = (tt // ne) * 16
                e0 = (tt % ne) * 16
                rvec = it16 + r0
                for k0 in range(0, 16, 4):
                    evs = [rotvecs[k0 + j] + e0 for j in range(4)]
                    vals = [plsc.load_gather(gi, [evs[j], rvec])
                            for j in range(4)]
                    for j in range(4):
                        plsc.store_scatter(go, [rvec, evs[j]], vals[j])
                return carry

            lax.fori_loop(0, (ncol // 16) * ne, tbody, 0)

        full = lambda t: unit(t) < n_units - 1
        live = lambda t: unit(t) < n_units

        @pl.when(full(0))
        def _():
            in_copy(0, 0).start()

        def step(t, b):
            @pl.when(full(t))
            def _():
                in_copy(t, b).wait()

            @pl.when(full(t + 1))
            def _():
                in_copy(t + 1, 1 - b).start()

            @pl.when(t >= 2)
            def _():
                @pl.when(full(t - 2))
                def _():
                    store_copy(t - 2, b).wait()

            @pl.when(full(t))
            def _():
                transpose(gin.at[b], gout.at[b], 128)
                store_copy(t, b).start()

            @pl.when(live(t) & jnp.logical_not(full(t)))
            def _():
                pltpu.sync_copy(tail_hbm, gtail)
                tc = pltpu.make_async_copy(
                    gtail, out_hbm.at[pl.ds(unit(t) * 128, vtail)], sem_t)
                tc.start()
                tc.wait()

        def body(g, carry):
            step(2 * g, 0)
            step(2 * g + 1, 1)
            return carry

        n_pairs = n_t // 2
        lax.fori_loop(0, n_pairs, body, 0)
        if n_t % 2:
            step(n_t - 1, (n_t - 1) % 2)
        for tt in (n_t - 2, n_t - 1):
            @pl.when(full(tt))
            def _():
                store_copy(tt, tt % 2).wait()

    return k1


def kernel(input_ids, table):
    bsz, seq = input_ids.shape
    v, d = table.shape
    ids_t = input_ids.T.astype(jnp.int32)
    tail = jnp.pad(table[v - v % 128:], ((0, 0), (0, 128 - d)))
    table_pad = _build_pad(v, d)(table.T, tail)
    out_t = _build(bsz, seq, v, d)(table_pad, ids_t)
    return (out_t.transpose(2, 0, 1),)
